# Initial kernel scaffold; baseline (speedup 1.0000x reference)
import jax
import jax.numpy as jnp
from jax.experimental import pallas as pl


def kernel(x, edge_attr, edge_index, pos, params):
    raise NotImplementedError("write your pallas kernel here")


# speedup vs baseline: 1.0213x; 1.0213x over previous
import jax
import jax.numpy as jnp
from jax.experimental import pallas as pl

N = 50000
E = 800000
H = 64
NUM_LAYERS = 3
BN = 2000
BE = 4000


def _ln(h, g, beta):
    mu = jnp.mean(h, axis=-1, keepdims=True)
    d = h - mu
    var = jnp.mean(d * d, axis=-1, keepdims=True)
    return d * jax.lax.rsqrt(var + 1e-5) * g + beta


def _softplus(x):
    return jnp.maximum(x, 0.0) + jnp.log1p(jnp.exp(-jnp.abs(x)))


def _mm(a, w):
    return jax.lax.dot_general(a, w, (((1,), (0,)), ((), ())),
                               preferred_element_type=jnp.float32)


def _row_call(fn, n_rows, blk, outs, *args):
    grid = (n_rows // blk,)
    in_specs = []
    for a in args:
        if a.shape[0] == n_rows:
            in_specs.append(pl.BlockSpec((blk, a.shape[1]), lambda i: (i, 0)))
        else:
            in_specs.append(pl.BlockSpec(a.shape, lambda i, nd=a.ndim: (0,) * nd))
    out_specs = [pl.BlockSpec((blk, c), lambda i: (i, 0)) for c in outs]
    out_shape = [jax.ShapeDtypeStruct((n_rows, c), jnp.float32) for c in outs]
    if len(outs) == 1:
        out_specs, out_shape = out_specs[0], out_shape[0]
    return pl.pallas_call(
        fn, grid=grid, in_specs=in_specs, out_specs=out_specs,
        out_shape=out_shape)(*args)


def _node_encode_k(x_ref, pos_ref, w1, b1, w2, b2, g, beta, pf1, pb1, pf2,
                   pb2, out_ref):
    h = _mm(jnp.maximum(_mm(x_ref[...], w1[...]) + b1[...], 0.0), w2[...]) \
        + b2[...]
    h = _ln(h, g[...], beta[...])
    pf = _mm(jnp.maximum(_mm(pos_ref[...], pf1[...]) + pb1[...], 0.0),
             pf2[...]) + pb2[...]
    out_ref[...] = h + jnp.concatenate([pf, pf, pf, pf], axis=1)


def _edge_encode_k(ea_ref, w1, b1, w2, b2, g, beta, l0w, l0b, l1w, l1b, l2w,
                   l2b, o0, o1, o2):
    e = _mm(jnp.maximum(_mm(ea_ref[...], w1[...]) + b1[...], 0.0), w2[...]) \
        + b2[...]
    e = _ln(e, g[...], beta[...])
    o0[...] = _mm(e, l0w[...]) + l0b[...]
    o1[...] = _mm(e, l1w[...]) + l1b[...]
    o2[...] = _mm(e, l2w[...]) + l2b[...]


def _msg_k(hs_ref, el_ref, out_ref):
    out_ref[...] = jnp.maximum(hs_ref[...] + el_ref[...], 0.0)


def _layer_k(h_ref, agg_ref, w1, b1, w2, b2, g, beta, out_ref):
    z = h_ref[...] + agg_ref[...]
    z = _mm(jnp.maximum(_mm(z, w1[...]) + b1[...], 0.0), w2[...]) + b2[...]
    out_ref[...] = jnp.maximum(_ln(z, g[...], beta[...]), 0.0)


def _heads_k(h_ref, d1w, d1b, d2w, d2b, s1w, s1b, s2w, s2b, f1w, f1b, f2w,
             f2b, out_ref):
    h = h_ref[...]
    disp = _mm(jnp.maximum(_mm(h, d1w[...]) + d1b[...], 0.0), d2w[...]) \
        + d2b[...]
    st = _softplus(_mm(jnp.maximum(_mm(h, s1w[...]) + s1b[...], 0.0),
                       s2w[...]) + s2b[...])
    sf = _softplus(_mm(jnp.maximum(_mm(h, f1w[...]) + f1b[...], 0.0),
                       f2w[...]) + f2b[...])
    out_ref[...] = jnp.concatenate(
        [disp, st, sf, jnp.zeros_like(st), jnp.zeros_like(st),
         jnp.zeros_like(st)], axis=1)


def _edge_stress_k(pk_ref, out_ref):
    pk = pk_ref[...]
    du = pk[:, 0:3] - pk[:, 6:9]
    dx = pk[:, 3:6] - pk[:, 9:12]
    dist = jnp.sqrt(jnp.sum(dx * dx, axis=1, keepdims=True)) + 1e-8
    strain = jnp.sqrt(jnp.sum(du * du, axis=1, keepdims=True)) / dist
    out_ref[...] = 2.1e11 * strain


def _vm_k(ns_ref, cnt_ref, out_ref):
    out_ref[...] = ns_ref[...] / (cnt_ref[...] + 1e-8)


def kernel(x, edge_attr, edge_index, pos, params):
    p = params
    r = lambda v: v.reshape(1, -1)

    h = _row_call(_node_encode_k, N, BN, (H,), x, pos,
                  p['ne1_W'], r(p['ne1_b']), p['ne2_W'], r(p['ne2_b']),
                  r(p['ne_g']), r(p['ne_beta']),
                  p['pf1_W'], r(p['pf1_b']), p['pf2_W'], r(p['pf2_b']))

    els = _row_call(_edge_encode_k, E, BE, (H, H, H), edge_attr,
                    p['ee1_W'], r(p['ee1_b']), p['ee2_W'], r(p['ee2_b']),
                    r(p['ee_g']), r(p['ee_beta']),
                    p['lin0_W'], r(p['lin0_b']), p['lin1_W'], r(p['lin1_b']),
                    p['lin2_W'], r(p['lin2_b']))

    src = edge_index[0]
    dst = edge_index[1]

    for i in range(NUM_LAYERS):
        hs = jnp.take(h, src, axis=0)
        msg = _row_call(_msg_k, E, BE, (H,), hs, els[i])
        agg = jax.ops.segment_sum(msg, dst, num_segments=N)
        h = _row_call(_layer_k, N, BN, (H,), h, agg,
                      p['conv%d_1_W' % i], r(p['conv%d_1_b' % i]),
                      p['conv%d_2_W' % i], r(p['conv%d_2_b' % i]),
                      r(p['conv%d_g' % i]), r(p['conv%d_beta' % i]))

    heads = _row_call(_heads_k, N, BN, (8,), h,
                      p['dh1_W'], r(p['dh1_b']), p['dh2_W'], r(p['dh2_b']),
                      p['sh1_W'], r(p['sh1_b']), p['sh2_W'], r(p['sh2_b']),
                      p['sf1_W'], r(p['sf1_b']), p['sf2_W'], r(p['sf2_b']))
    disp = heads[:, 0:3]
    stress = heads[:, 3:4]
    sf = heads[:, 4:5]

    np_pack = jnp.concatenate([disp, pos], axis=1)
    pk = jnp.concatenate([jnp.take(np_pack, dst, axis=0),
                          jnp.take(np_pack, src, axis=0)], axis=1)
    es = _row_call(_edge_stress_k, E, BE, (1,), pk)

    node_stress = jnp.zeros((N, 1), jnp.float32).at[src].add(es).at[dst].add(es)
    ones = jnp.ones_like(es)
    cnt = jnp.zeros((N, 1), jnp.float32).at[src].add(ones).at[dst].add(ones)
    von_mises = _row_call(_vm_k, N, BN, (1,), node_stress, cnt)

    return disp, stress, von_mises, sf, h
